# Initial kernel scaffold; baseline (speedup 1.0000x reference)
import jax
import jax.numpy as jnp
from jax.experimental import pallas as pl


def kernel(inputs, edges, gcn_w, gcn_b, fc1_w, fc1_b, fc2_w, fc2_b):
    raise NotImplementedError("write your pallas kernel here")


# speedup vs baseline: 26.2726x; 26.2726x over previous
import functools

import jax
import jax.numpy as jnp
from jax import lax
from jax.experimental import pallas as pl
from jax.experimental.pallas import tpu as pltpu
from jax.experimental.pallas import tpu_sc as plsc

B = 102
F_IN = 978
N = B * F_IN
E = N * 32

NC, NS = 2, 16
NW = NC * NS
LANES = 16

CHUNK_ROWS = 9
CHUNK = CHUNK_ROWS * 128
NCHUNKS = E // CHUNK
BASE_CHUNKS = NCHUNKS // NW
EXTRA = NCHUNKS - BASE_CHUNKS * NW

NPAD = 99840
SLICE = NPAD // NS


def _sc_body(x_hbm, edges_hbm, out_hbm, x_v, src_v, dst_v, vals_v, zero_v, acc_s):
    cid = lax.axis_index("c")
    sid = lax.axis_index("s")
    wid = sid * NC + cid

    pltpu.sync_copy(x_hbm, x_v)

    def _zero(k, _):
        zero_v[pl.ds(k * LANES, LANES)] = jnp.zeros((LANES,), jnp.float32)
        return 0
    lax.fori_loop(0, SLICE // LANES, _zero, 0, unroll=8)
    pltpu.sync_copy(zero_v, acc_s.at[pl.ds(sid * SLICE, SLICE)])
    plsc.subcore_barrier()

    cstart = wid * BASE_CHUNKS + jnp.minimum(wid, EXTRA)
    ccount = BASE_CHUNKS + (wid < EXTRA).astype(jnp.int32)

    def _chunk(i, _):
        off = (cstart + i) * CHUNK
        pltpu.sync_copy(edges_hbm.at[pl.ds(off, CHUNK)], src_v)
        for j in range(CHUNK_ROWS):
            pltpu.sync_copy(edges_hbm.at[pl.ds(E + off + j * 128, 128)],
                            dst_v.at[j])
        for k in range(CHUNK // LANES):
            idx = src_v[pl.ds(k * LANES, LANES)]
            vals_v[k // 8, pl.ds((k % 8) * LANES, LANES)] = \
                plsc.load_gather(x_v, [idx])
        for j in range(CHUNK_ROWS):
            pltpu.sync_copy(vals_v.at[j], acc_s.at[dst_v.at[j]], add=True)
        return 0

    lax.fori_loop(0, ccount, _chunk, 0)
    plsc.subcore_barrier()

    pltpu.sync_copy(acc_s.at[pl.ds(sid * SLICE, SLICE)], zero_v)
    pltpu.sync_copy(zero_v, out_hbm.at[pl.ds(cid * NPAD + sid * SLICE, SLICE)])


@jax.jit
def _sc_scatter(x_pad, edges_r):
    mesh = plsc.VectorSubcoreMesh(core_axis_name="c", subcore_axis_name="s",
                                  num_cores=NC, num_subcores=NS)
    f = pl.kernel(
        _sc_body,
        out_type=jax.ShapeDtypeStruct((NC * NPAD,), jnp.float32),
        mesh=mesh,
        scratch_types=[
            pltpu.VMEM((NPAD,), jnp.float32),
            pltpu.VMEM((CHUNK,), jnp.int32),
            pltpu.VMEM((CHUNK_ROWS, 128), jnp.int32),
            pltpu.VMEM((CHUNK_ROWS, 128), jnp.float32),
            pltpu.VMEM((SLICE,), jnp.float32),
            pltpu.VMEM_SHARED((NPAD,), jnp.float32),
        ],
        compiler_params=pltpu.CompilerParams(needs_layout_passes=False),
    )
    return f(x_pad, edges_r)


def _mlp_body(a0, a1, gw, gb, w1, b1, w2, b2, o_ref):
    h = (a0[...] + a1[...]) * gw[0, 0] + gb[0, 0]
    h = jnp.maximum(h, 0.0)
    h1 = lax.dot_general(h, w1[...], (((1,), (1,)), ((), ())),
                         preferred_element_type=jnp.float32) + b1[...]
    h1 = jnp.maximum(h1, 0.0)
    o_ref[...] = lax.dot_general(h1, w2[...], (((1,), (1,)), ((), ())),
                                 preferred_element_type=jnp.float32) + b2[...]


@jax.jit
def _tc_mlp(a0, a1, gw, gb, w1, b1, w2, b2):
    smem = pl.BlockSpec(memory_space=pltpu.SMEM)
    return pl.pallas_call(
        _mlp_body,
        out_shape=jax.ShapeDtypeStruct((B, 100), jnp.float32),
        in_specs=[pl.BlockSpec((B, F_IN), lambda: (0, 0)),
                  pl.BlockSpec((B, F_IN), lambda: (0, 0)),
                  smem, smem,
                  pl.BlockSpec((2048, F_IN), lambda: (0, 0)),
                  pl.BlockSpec((1, 2048), lambda: (0, 0)),
                  pl.BlockSpec((100, 2048), lambda: (0, 0)),
                  pl.BlockSpec((1, 100), lambda: (0, 0))],
        out_specs=pl.BlockSpec((B, 100), lambda: (0, 0)),
    )(a0, a1, gw, gb, w1, b1, w2, b2)


def kernel(inputs, edges, gcn_w, gcn_b, fc1_w, fc1_b, fc2_w, fc2_b):
    x = inputs.reshape(-1)
    x_pad = jnp.pad(x, (0, NPAD - N))
    edges_r = edges.reshape(2 * E)
    acc = _sc_scatter(x_pad, edges_r)
    a0 = acc[:N].reshape(B, F_IN)
    a1 = acc[NPAD:NPAD + N].reshape(B, F_IN)
    return _tc_mlp(a0, a1,
                   gcn_w.reshape(1, 1), gcn_b.reshape(1, 1),
                   fc1_w, fc1_b.reshape(1, 2048),
                   fc2_w, fc2_b.reshape(1, 100))


# speedup vs baseline: 40.1698x; 1.5290x over previous
import functools

import jax
import jax.numpy as jnp
from jax import lax
from jax.experimental import pallas as pl
from jax.experimental.pallas import tpu as pltpu
from jax.experimental.pallas import tpu_sc as plsc

B = 102
F_IN = 978
N = B * F_IN
E = N * 32

NC, NS = 2, 16
NW = NC * NS
LANES = 16

CHUNK_ROWS = 9
CHUNK = CHUNK_ROWS * 128
NCHUNKS = E // CHUNK
BASE_CHUNKS = NCHUNKS // NW
EXTRA = NCHUNKS - BASE_CHUNKS * NW

NPAD = 99840
SLICE = NPAD // NS


def _sc_body(x_hbm, edges_hbm, out_hbm, x_v, src_v, dst_v, vals_v, zero_v, acc_s):
    cid = lax.axis_index("c")
    sid = lax.axis_index("s")
    wid = sid * NC + cid

    pltpu.sync_copy(x_hbm, x_v)

    def _zero(k, _):
        zero_v[pl.ds(k * LANES, LANES)] = jnp.zeros((LANES,), jnp.float32)
        return 0
    lax.fori_loop(0, SLICE // LANES, _zero, 0, unroll=8)
    pltpu.sync_copy(zero_v, acc_s.at[pl.ds(sid * SLICE, SLICE)])
    plsc.subcore_barrier()

    cstart = wid * BASE_CHUNKS + jnp.minimum(wid, EXTRA)
    ccount = BASE_CHUNKS + (wid < EXTRA).astype(jnp.int32)

    def _chunk(i, _):
        off = (cstart + i) * CHUNK
        pltpu.sync_copy(edges_hbm.at[pl.ds(off, CHUNK)], src_v)
        pltpu.sync_copy(edges_hbm.at[pl.ds(E + off, CHUNK)], dst_v)
        for k in range(CHUNK // LANES):
            idx = src_v[pl.ds(k * LANES, LANES)]
            vals_v[pl.ds(k * LANES, LANES)] = plsc.load_gather(x_v, [idx])
        pltpu.sync_copy(vals_v, acc_s.at[dst_v], add=True)
        return 0

    lax.fori_loop(0, ccount, _chunk, 0)
    plsc.subcore_barrier()

    pltpu.sync_copy(acc_s.at[pl.ds(sid * SLICE, SLICE)], zero_v)
    pltpu.sync_copy(zero_v, out_hbm.at[pl.ds(cid * NPAD + sid * SLICE, SLICE)])


@jax.jit
def _sc_scatter(x_pad, edges_r):
    mesh = plsc.VectorSubcoreMesh(core_axis_name="c", subcore_axis_name="s",
                                  num_cores=NC, num_subcores=NS)
    f = pl.kernel(
        _sc_body,
        out_type=jax.ShapeDtypeStruct((NC * NPAD,), jnp.float32),
        mesh=mesh,
        scratch_types=[
            pltpu.VMEM((NPAD,), jnp.float32),
            pltpu.VMEM((CHUNK,), jnp.int32),
            pltpu.VMEM((CHUNK,), jnp.int32),
            pltpu.VMEM((CHUNK,), jnp.float32),
            pltpu.VMEM((SLICE,), jnp.float32),
            pltpu.VMEM_SHARED((NPAD,), jnp.float32),
        ],
        compiler_params=pltpu.CompilerParams(needs_layout_passes=False),
    )
    return f(x_pad, edges_r)


def _mlp_body(a0, a1, gw, gb, w1, b1, w2, b2, o_ref):
    h = (a0[...] + a1[...]) * gw[0, 0] + gb[0, 0]
    h = jnp.maximum(h, 0.0)
    h1 = lax.dot_general(h, w1[...], (((1,), (1,)), ((), ())),
                         preferred_element_type=jnp.float32) + b1[...]
    h1 = jnp.maximum(h1, 0.0)
    o_ref[...] = lax.dot_general(h1, w2[...], (((1,), (1,)), ((), ())),
                                 preferred_element_type=jnp.float32) + b2[...]


@jax.jit
def _tc_mlp(a0, a1, gw, gb, w1, b1, w2, b2):
    smem = pl.BlockSpec(memory_space=pltpu.SMEM)
    return pl.pallas_call(
        _mlp_body,
        out_shape=jax.ShapeDtypeStruct((B, 100), jnp.float32),
        in_specs=[pl.BlockSpec((B, F_IN), lambda: (0, 0)),
                  pl.BlockSpec((B, F_IN), lambda: (0, 0)),
                  smem, smem,
                  pl.BlockSpec((2048, F_IN), lambda: (0, 0)),
                  pl.BlockSpec((1, 2048), lambda: (0, 0)),
                  pl.BlockSpec((100, 2048), lambda: (0, 0)),
                  pl.BlockSpec((1, 100), lambda: (0, 0))],
        out_specs=pl.BlockSpec((B, 100), lambda: (0, 0)),
    )(a0, a1, gw, gb, w1, b1, w2, b2)


def kernel(inputs, edges, gcn_w, gcn_b, fc1_w, fc1_b, fc2_w, fc2_b):
    x = inputs.reshape(-1)
    x_pad = jnp.pad(x, (0, NPAD - N))
    edges_r = edges.reshape(2 * E)
    acc = _sc_scatter(x_pad, edges_r)
    a0 = acc[:N].reshape(B, F_IN)
    a1 = acc[NPAD:NPAD + N].reshape(B, F_IN)
    return _tc_mlp(a0, a1,
                   gcn_w.reshape(1, 1), gcn_b.reshape(1, 1),
                   fc1_w, fc1_b.reshape(1, 2048),
                   fc2_w, fc2_b.reshape(1, 100))


# speedup vs baseline: 48.6953x; 1.2122x over previous
import functools

import jax
import jax.numpy as jnp
from jax import lax
from jax.experimental import pallas as pl
from jax.experimental.pallas import tpu as pltpu
from jax.experimental.pallas import tpu_sc as plsc

B = 102
F_IN = 978
N = B * F_IN
E = N * 32

NC, NS = 2, 16
NW = NC * NS
LANES = 16

CHUNK_ROWS = 9
CHUNK = CHUNK_ROWS * 128
NCHUNKS = E // CHUNK
BASE_CHUNKS = NCHUNKS // NW
EXTRA = NCHUNKS - BASE_CHUNKS * NW

NPAD = 99840
SLICE = NPAD // NS


NBUF = 4


def _sc_body(x_hbm, edges_hbm, out_hbm, x_v, src_v, dst_v, vals_v, zero_v,
             acc_s, lsems, ssems):
    cid = lax.axis_index("c")
    sid = lax.axis_index("s")
    wid = sid * NC + cid

    cstart = wid * BASE_CHUNKS + jnp.minimum(wid, EXTRA)
    ccount = BASE_CHUNKS + (wid < EXTRA).astype(jnp.int32)
    cend = cstart + ccount

    def _loads(c, b):
        off = c * CHUNK
        pltpu.make_async_copy(edges_hbm.at[pl.ds(off, CHUNK)],
                              src_v[b], lsems[b]).start()
        pltpu.make_async_copy(edges_hbm.at[pl.ds(E + off, CHUNK)],
                              dst_v[b], lsems[b]).start()

    def _wait_loads(b):
        pltpu.make_async_copy(edges_hbm.at[pl.ds(0, CHUNK)],
                              src_v[b], lsems[b]).wait()
        pltpu.make_async_copy(edges_hbm.at[pl.ds(0, CHUNK)],
                              dst_v[b], lsems[b]).wait()

    def _scatter(b):
        return pltpu.make_async_copy(vals_v[b], acc_s.at[dst_v[b]],
                                     ssems[b])

    for b in range(NBUF):
        _loads(cstart + b, b)

    pltpu.sync_copy(x_hbm, x_v)

    def _zero(k, _):
        zero_v[pl.ds(k * LANES, LANES)] = jnp.zeros((LANES,), jnp.float32)
        return 0
    lax.fori_loop(0, SLICE // LANES, _zero, 0, unroll=8)
    pltpu.sync_copy(zero_v, acc_s.at[pl.ds(sid * SLICE, SLICE)])
    plsc.subcore_barrier()

    def _round(r, _):
        base = cstart + r * NBUF
        for b in range(NBUF):
            @pl.when(base + b < cend)
            def _():
                _wait_loads(b)
                for k in range(CHUNK // LANES):
                    idx = src_v[b][pl.ds(k * LANES, LANES)]
                    vals_v[b][pl.ds(k * LANES, LANES)] = \
                        plsc.load_gather(x_v, [idx])
                _scatter(b).start(add=True)
        for b in range(NBUF):
            @pl.when(base + NBUF + b < cend)
            def _():
                _scatter(b).wait()
                _loads(base + NBUF + b, b)
        return 0

    nrounds = (ccount + NBUF - 1) // NBUF
    lax.fori_loop(0, nrounds, _round, 0)
    for b in range(NBUF):
        _scatter(b).wait()
    plsc.subcore_barrier()

    pltpu.sync_copy(acc_s.at[pl.ds(sid * SLICE, SLICE)], zero_v)
    pltpu.sync_copy(zero_v, out_hbm.at[pl.ds(cid * NPAD + sid * SLICE, SLICE)])


@jax.jit
def _sc_scatter(x_pad, edges_r):
    mesh = plsc.VectorSubcoreMesh(core_axis_name="c", subcore_axis_name="s",
                                  num_cores=NC, num_subcores=NS)
    f = pl.kernel(
        _sc_body,
        out_type=jax.ShapeDtypeStruct((NC * NPAD,), jnp.float32),
        mesh=mesh,
        scratch_types=[
            pltpu.VMEM((NPAD,), jnp.float32),
            [pltpu.VMEM((CHUNK,), jnp.int32)] * NBUF,
            [pltpu.VMEM((CHUNK,), jnp.int32)] * NBUF,
            [pltpu.VMEM((CHUNK,), jnp.float32)] * NBUF,
            pltpu.VMEM((SLICE,), jnp.float32),
            pltpu.VMEM_SHARED((NPAD,), jnp.float32),
            [pltpu.SemaphoreType.DMA] * NBUF,
            [pltpu.SemaphoreType.DMA] * NBUF,
        ],
        compiler_params=pltpu.CompilerParams(needs_layout_passes=False),
    )
    return f(x_pad, edges_r)


def _mlp_body(a0, a1, gw, gb, w1, b1, w2, b2, o_ref):
    h = (a0[...] + a1[...]) * gw[0, 0] + gb[0, 0]
    h = jnp.maximum(h, 0.0)
    h1 = lax.dot_general(h, w1[...], (((1,), (1,)), ((), ())),
                         preferred_element_type=jnp.float32) + b1[...]
    h1 = jnp.maximum(h1, 0.0)
    o_ref[...] = lax.dot_general(h1, w2[...], (((1,), (1,)), ((), ())),
                                 preferred_element_type=jnp.float32) + b2[...]


@jax.jit
def _tc_mlp(a0, a1, gw, gb, w1, b1, w2, b2):
    smem = pl.BlockSpec(memory_space=pltpu.SMEM)
    return pl.pallas_call(
        _mlp_body,
        out_shape=jax.ShapeDtypeStruct((B, 100), jnp.float32),
        in_specs=[pl.BlockSpec((B, F_IN), lambda: (0, 0)),
                  pl.BlockSpec((B, F_IN), lambda: (0, 0)),
                  smem, smem,
                  pl.BlockSpec((2048, F_IN), lambda: (0, 0)),
                  pl.BlockSpec((1, 2048), lambda: (0, 0)),
                  pl.BlockSpec((100, 2048), lambda: (0, 0)),
                  pl.BlockSpec((1, 100), lambda: (0, 0))],
        out_specs=pl.BlockSpec((B, 100), lambda: (0, 0)),
    )(a0, a1, gw, gb, w1, b1, w2, b2)


def kernel(inputs, edges, gcn_w, gcn_b, fc1_w, fc1_b, fc2_w, fc2_b):
    x = inputs.reshape(-1)
    x_pad = jnp.pad(x, (0, NPAD - N))
    edges_r = edges.reshape(2 * E)
    acc = _sc_scatter(x_pad, edges_r)
    a0 = acc[:N].reshape(B, F_IN)
    a1 = acc[NPAD:NPAD + N].reshape(B, F_IN)
    return _tc_mlp(a0, a1,
                   gcn_w.reshape(1, 1), gcn_b.reshape(1, 1),
                   fc1_w, fc1_b.reshape(1, 2048),
                   fc2_w, fc2_b.reshape(1, 100))


# speedup vs baseline: 356.7081x; 7.3253x over previous
import functools

import jax
import jax.numpy as jnp
from jax import lax
from jax.experimental import pallas as pl
from jax.experimental.pallas import tpu as pltpu
from jax.experimental.pallas import tpu_sc as plsc

B = 102
F_IN = 978
N = B * F_IN
E = N * 32

NC, NS = 2, 16
NW = NC * NS
LANES = 16

CHUNK_ROWS = 9
CHUNK = CHUNK_ROWS * 128
NCHUNKS = E // CHUNK
BASE_CHUNKS = NCHUNKS // NW
EXTRA = NCHUNKS - BASE_CHUNKS * NW

NPAD = 99840
SLICE = NPAD // NS


NBUF = 4


def _sc_body(x_hbm, edges_hbm, out_hbm, x_v, ev_v, dst_v, vals_v,
             zero_v, acc_s, lsems, ssems):
    cid = lax.axis_index("c")
    sid = lax.axis_index("s")
    wid = sid * NC + cid

    cstart = wid * BASE_CHUNKS + jnp.minimum(wid, EXTRA)
    ccount = BASE_CHUNKS + (wid < EXTRA).astype(jnp.int32)
    cend = cstart + ccount

    def _loads(c, b):
        off = c * CHUNK
        pltpu.make_async_copy(edges_hbm.at[:, pl.ds(off, CHUNK)],
                              ev_v[b], lsems[b]).start()

    def _wait_loads(b):
        pltpu.make_async_copy(edges_hbm.at[:, pl.ds(0, CHUNK)],
                              ev_v[b], lsems[b]).wait()

    def _scatter(b):
        return pltpu.make_async_copy(vals_v[b], acc_s.at[dst_v[b]],
                                     ssems[b])

    for b in range(NBUF):
        _loads(cstart + b, b)

    pltpu.sync_copy(x_hbm, x_v)

    def _zero(k, _):
        zero_v[pl.ds(k * LANES, LANES)] = jnp.zeros((LANES,), jnp.float32)
        return 0
    lax.fori_loop(0, SLICE // LANES, _zero, 0, unroll=8)
    pltpu.sync_copy(zero_v, acc_s.at[pl.ds(sid * SLICE, SLICE)])
    plsc.subcore_barrier()

    def _round(r, _):
        base = cstart + r * NBUF
        for b in range(NBUF):
            @pl.when(base + b < cend)
            def _():
                _wait_loads(b)
                for k0 in range(0, CHUNK // LANES, 8):
                    sls = [pl.ds((k0 + i) * LANES, LANES) for i in range(8)]
                    srcs = [ev_v[b][0, s] for s in sls]
                    dsts = [ev_v[b][1, s] for s in sls]
                    vals = [plsc.load_gather(x_v, [ix]) for ix in srcs]
                    for i in range(8):
                        vals_v[b][sls[i]] = vals[i]
                        dst_v[b][sls[i]] = dsts[i]
                _scatter(b).start(add=True)
        for b in range(NBUF):
            @pl.when(base + NBUF + b < cend)
            def _():
                _scatter(b).wait()
                _loads(base + NBUF + b, b)
        return 0

    nrounds = (ccount + NBUF - 1) // NBUF
    lax.fori_loop(0, nrounds, _round, 0)
    for b in range(NBUF):
        _scatter(b).wait()
    plsc.subcore_barrier()

    pltpu.sync_copy(acc_s.at[pl.ds(sid * SLICE, SLICE)], zero_v)
    pltpu.sync_copy(zero_v, out_hbm.at[pl.ds(cid * NPAD + sid * SLICE, SLICE)])


@jax.jit
def _sc_scatter(x_pad, edges):
    mesh = plsc.VectorSubcoreMesh(core_axis_name="c", subcore_axis_name="s",
                                  num_cores=NC, num_subcores=NS)
    f = pl.kernel(
        _sc_body,
        out_type=jax.ShapeDtypeStruct((NC * NPAD,), jnp.float32),
        mesh=mesh,
        scratch_types=[
            pltpu.VMEM((NPAD,), jnp.float32),
            [pltpu.VMEM((2, CHUNK), jnp.int32)] * NBUF,
            [pltpu.VMEM((CHUNK,), jnp.int32)] * NBUF,
            [pltpu.VMEM((CHUNK,), jnp.float32)] * NBUF,
            pltpu.VMEM((SLICE,), jnp.float32),
            pltpu.VMEM_SHARED((NPAD,), jnp.float32),
            [pltpu.SemaphoreType.DMA] * NBUF,
            [pltpu.SemaphoreType.DMA] * NBUF,
        ],
        compiler_params=pltpu.CompilerParams(needs_layout_passes=False),
    )
    return f(x_pad, edges)


def _mlp_body(a0, a1, gw, gb, w1, b1, w2, b2, o_ref):
    h = (a0[...] + a1[...]) * gw[0, 0] + gb[0, 0]
    h = jnp.maximum(h, 0.0)
    h1 = lax.dot_general(h, w1[...], (((1,), (0,)), ((), ())),
                         preferred_element_type=jnp.float32) + b1[...]
    h1 = jnp.maximum(h1, 0.0)
    o_ref[...] = lax.dot_general(h1, w2[...], (((1,), (1,)), ((), ())),
                                 preferred_element_type=jnp.float32) + b2[...]


@jax.jit
def _tc_mlp(a0, a1, gw, gb, w1, b1, w2, b2):
    smem = pl.BlockSpec(memory_space=pltpu.SMEM)
    return pl.pallas_call(
        _mlp_body,
        out_shape=jax.ShapeDtypeStruct((B, 100), jnp.float32),
        in_specs=[pl.BlockSpec((B, F_IN), lambda: (0, 0)),
                  pl.BlockSpec((B, F_IN), lambda: (0, 0)),
                  smem, smem,
                  pl.BlockSpec((F_IN, 2048), lambda: (0, 0)),
                  pl.BlockSpec((1, 2048), lambda: (0, 0)),
                  pl.BlockSpec((100, 2048), lambda: (0, 0)),
                  pl.BlockSpec((1, 100), lambda: (0, 0))],
        out_specs=pl.BlockSpec((B, 100), lambda: (0, 0)),
    )(a0, a1, gw, gb, w1, b1, w2, b2)


def kernel(inputs, edges, gcn_w, gcn_b, fc1_w, fc1_b, fc2_w, fc2_b):
    x = inputs.reshape(-1)
    x_pad = jnp.pad(x, (0, NPAD - N))
    acc = _sc_scatter(x_pad, edges)
    a0 = acc[:N].reshape(B, F_IN)
    a1 = acc[NPAD:NPAD + N].reshape(B, F_IN)
    return _tc_mlp(a0, a1,
                   gcn_w.reshape(1, 1), gcn_b.reshape(1, 1),
                   fc1_w.T, fc1_b.reshape(1, 2048),
                   fc2_w, fc2_b.reshape(1, 100))


# speedup vs baseline: 357.4308x; 1.0020x over previous
import jax
import jax.numpy as jnp
from jax import lax
from jax.experimental import pallas as pl
from jax.experimental.pallas import tpu as pltpu
from jax.experimental.pallas import tpu_sc as plsc

B = 102
F_IN = 978
N = B * F_IN
E = N * 32

NC, NS = 2, 16
NW = NC * NS
LANES = 16

CHUNK_ROWS = 9
CHUNK = CHUNK_ROWS * 128
NCHUNKS = E // CHUNK
BASE_CHUNKS = NCHUNKS // NW
EXTRA = NCHUNKS - BASE_CHUNKS * NW

NPAD = 99840
SLICE = NPAD // NS


NBUF = 4


def _sc_body(x_hbm, edges_hbm, out_hbm, x_v, ev_v, dst_v, vals_v,
             zero_v, acc_s, lsems, ssems):
    cid = lax.axis_index("c")
    sid = lax.axis_index("s")
    wid = sid * NC + cid

    cstart = wid * BASE_CHUNKS + jnp.minimum(wid, EXTRA)
    ccount = BASE_CHUNKS + (wid < EXTRA).astype(jnp.int32)
    cend = cstart + ccount

    def _loads(c, b):
        off = c * CHUNK
        pltpu.make_async_copy(edges_hbm.at[:, pl.ds(off, CHUNK)],
                              ev_v[b], lsems[b]).start()

    def _wait_loads(b):
        pltpu.make_async_copy(edges_hbm.at[:, pl.ds(0, CHUNK)],
                              ev_v[b], lsems[b]).wait()

    def _scatter(b):
        return pltpu.make_async_copy(vals_v[b], acc_s.at[dst_v[b]],
                                     ssems[b])

    for b in range(NBUF):
        _loads(cstart + b, b)

    pltpu.sync_copy(x_hbm, x_v)

    def _zero(k, _):
        zero_v[pl.ds(k * LANES, LANES)] = jnp.zeros((LANES,), jnp.float32)
        return 0
    lax.fori_loop(0, SLICE // LANES, _zero, 0, unroll=8)
    pltpu.sync_copy(zero_v, acc_s.at[pl.ds(sid * SLICE, SLICE)])
    plsc.subcore_barrier()

    def _round(r, _):
        base = cstart + r * NBUF
        for b in range(NBUF):
            @pl.when(base + b < cend)
            def _():
                _wait_loads(b)
                for k0 in range(0, CHUNK // LANES, 8):
                    sls = [pl.ds((k0 + i) * LANES, LANES) for i in range(8)]
                    srcs = [ev_v[b][0, s] for s in sls]
                    dsts = [ev_v[b][1, s] for s in sls]
                    vals = [plsc.load_gather(x_v, [ix]) for ix in srcs]
                    for i in range(8):
                        vals_v[b][sls[i]] = vals[i]
                        dst_v[b][sls[i]] = dsts[i]
                _scatter(b).start(add=True)
        for b in range(NBUF):
            @pl.when(base + NBUF + b < cend)
            def _():
                _scatter(b).wait()
                _loads(base + NBUF + b, b)
        return 0

    nrounds = (ccount + NBUF - 1) // NBUF
    lax.fori_loop(0, nrounds, _round, 0)
    for b in range(NBUF):
        _scatter(b).wait()
    plsc.subcore_barrier()

    pltpu.sync_copy(acc_s.at[pl.ds(sid * SLICE, SLICE)], zero_v)
    pltpu.sync_copy(zero_v, out_hbm.at[pl.ds(cid * NPAD + sid * SLICE, SLICE)])


@jax.jit
def _sc_scatter(x_pad, edges):
    mesh = plsc.VectorSubcoreMesh(core_axis_name="c", subcore_axis_name="s",
                                  num_cores=NC, num_subcores=NS)
    f = pl.kernel(
        _sc_body,
        out_type=jax.ShapeDtypeStruct((NC * NPAD,), jnp.float32),
        mesh=mesh,
        scratch_types=[
            pltpu.VMEM((NPAD,), jnp.float32),
            [pltpu.VMEM((2, CHUNK), jnp.int32)] * NBUF,
            [pltpu.VMEM((CHUNK,), jnp.int32)] * NBUF,
            [pltpu.VMEM((CHUNK,), jnp.float32)] * NBUF,
            pltpu.VMEM((SLICE,), jnp.float32),
            pltpu.VMEM_SHARED((NPAD,), jnp.float32),
            [pltpu.SemaphoreType.DMA] * NBUF,
            [pltpu.SemaphoreType.DMA] * NBUF,
        ],
        compiler_params=pltpu.CompilerParams(needs_layout_passes=False),
    )
    return f(x_pad, edges)


def _mlp_body(a0, a1, gw, gb, w1, b1, w2, b2, o_ref):
    h = (a0[...] + a1[...]) * gw[0, 0] + gb[0, 0]
    h = jnp.maximum(h, 0.0)
    h1 = lax.dot_general(h, w1[...], (((1,), (0,)), ((), ())),
                         preferred_element_type=jnp.float32) + b1[...]
    h1 = jnp.maximum(h1, 0.0)
    o_ref[...] = lax.dot_general(h1, w2[...], (((1,), (1,)), ((), ())),
                                 preferred_element_type=jnp.float32) + b2[...]


@jax.jit
def _tc_mlp(a0, a1, gw, gb, w1, b1, w2, b2):
    smem = pl.BlockSpec(memory_space=pltpu.SMEM)
    return pl.pallas_call(
        _mlp_body,
        out_shape=jax.ShapeDtypeStruct((B, 100), jnp.float32),
        in_specs=[pl.BlockSpec((B, F_IN), lambda: (0, 0)),
                  pl.BlockSpec((B, F_IN), lambda: (0, 0)),
                  smem, smem,
                  pl.BlockSpec((F_IN, 2048), lambda: (0, 0)),
                  pl.BlockSpec((1, 2048), lambda: (0, 0)),
                  pl.BlockSpec((100, 2048), lambda: (0, 0)),
                  pl.BlockSpec((1, 100), lambda: (0, 0))],
        out_specs=pl.BlockSpec((B, 100), lambda: (0, 0)),
    )(a0, a1, gw, gb, w1, b1, w2, b2)


def kernel(inputs, edges, gcn_w, gcn_b, fc1_w, fc1_b, fc2_w, fc2_b):
    x = inputs.reshape(-1)
    x_pad = jnp.pad(x, (0, NPAD - N))
    acc = _sc_scatter(x_pad, edges)
    a0 = acc[:N].reshape(B, F_IN)
    a1 = acc[NPAD:NPAD + N].reshape(B, F_IN)
    return _tc_mlp(a0, a1,
                   gcn_w.reshape(1, 1), gcn_b.reshape(1, 1),
                   fc1_w.T, fc1_b.reshape(1, 2048),
                   fc2_w, fc2_b.reshape(1, 100))
